# baseline (speedup 1.0000x reference)
import functools

import jax
import jax.numpy as jnp
from jax.experimental import pallas as pl
from jax.experimental.pallas import tpu as pltpu

S, D, H, E, K, FFN = 2048, 768, 12, 8, 2, 2048
DH = D // H

_bf = jnp.bfloat16


def _dotT(a, b):
    return jax.lax.dot_general(
        a.astype(_bf), b.astype(_bf), (((1,), (1,)), ((), ())),
        preferred_element_type=jnp.float32)


def _ln_rows(x, g, b):
    m = jnp.mean(x, axis=-1, keepdims=True)
    v = jnp.mean((x - m) ** 2, axis=-1, keepdims=True)
    return (x - m) / jnp.sqrt(v + 1e-5) * g + b


def _qkv_body(xq_ref, xkv_ref, wq_ref, wk_ref, wv_ref, bq_ref, bk_ref, bv_ref,
              q_ref, k_ref, v_ref):
    q_ref[...] = _dotT(xq_ref[...], wq_ref[...]) + bq_ref[...]
    k_ref[...] = _dotT(xkv_ref[...], wk_ref[...]) + bk_ref[...]
    v_ref[...] = _dotT(xkv_ref[...], wv_ref[...]) + bv_ref[...]


def _qkv_proj(xq, xkv, wqkv, bqkv):
    RB = 256
    grid = (S // RB,)
    wq, wk, wv = wqkv[:D], wqkv[D:2 * D], wqkv[2 * D:]
    bq, bk, bv = bqkv[:D], bqkv[D:2 * D], bqkv[2 * D:]
    full_w = pl.BlockSpec((D, D), lambda i: (0, 0))
    full_b = pl.BlockSpec((D,), lambda i: (0,))
    row = pl.BlockSpec((RB, D), lambda i: (i, 0))
    return pl.pallas_call(
        _qkv_body,
        grid=grid,
        in_specs=[row, row, full_w, full_w, full_w, full_b, full_b, full_b],
        out_specs=[row, row, row],
        out_shape=[jax.ShapeDtypeStruct((S, D), jnp.float32)] * 3,
    )(xq, xkv, wq, wk, wv, bq, bk, bv)


def _attn_body(q_ref, k_ref, v_ref, o_ref):
    s = _dotT(q_ref[0], k_ref[0])
    m = jnp.max(s, axis=1, keepdims=True)
    p = jnp.exp((s - m) * 0.125)
    l = jnp.sum(p, axis=1, keepdims=True)
    o_ref[0] = jax.lax.dot_general(
        (p / l).astype(_bf), v_ref[0].astype(_bf), (((1,), (0,)), ((), ())),
        preferred_element_type=jnp.float32)


def _attention(q, k, v):
    QB = 1024
    grid = (H, S // QB)
    return pl.pallas_call(
        _attn_body,
        grid=grid,
        in_specs=[
            pl.BlockSpec((1, QB, DH), lambda h, qb: (h, qb, 0)),
            pl.BlockSpec((1, S, DH), lambda h, qb: (h, 0, 0)),
            pl.BlockSpec((1, S, DH), lambda h, qb: (h, 0, 0)),
        ],
        out_specs=pl.BlockSpec((1, QB, DH), lambda h, qb: (h, qb, 0)),
        out_shape=jax.ShapeDtypeStruct((H, S, DH), jnp.float32),
    )(q, k, v)


def _proj_ln_body(o_ref, res_ref, wo_ref, bo_ref, g_ref, b_ref, y_ref):
    y = _dotT(o_ref[...], wo_ref[...]) + bo_ref[...]
    x = res_ref[...] + y
    y_ref[...] = _ln_rows(x, g_ref[...], b_ref[...])


def _proj_ln(o, res, wo, bo, g, b):
    RB = 256
    row = pl.BlockSpec((RB, D), lambda i: (i, 0))
    return pl.pallas_call(
        _proj_ln_body,
        grid=(S // RB,),
        in_specs=[row, row,
                  pl.BlockSpec((D, D), lambda i: (0, 0)),
                  pl.BlockSpec((D,), lambda i: (0,)),
                  pl.BlockSpec((D,), lambda i: (0,)),
                  pl.BlockSpec((D,), lambda i: (0,))],
        out_specs=row,
        out_shape=jax.ShapeDtypeStruct((S, D), jnp.float32),
    )(o, res, wo, bo, g, b)


def _mha_block(xq, xkv, wqkv, bqkv, wo, bo, res, ln_g, ln_b):
    q, k, v = _qkv_proj(xq, xkv, wqkv, bqkv)
    qh = q.reshape(S, H, DH).transpose(1, 0, 2)
    kh = k.reshape(S, H, DH).transpose(1, 0, 2)
    vh = v.reshape(S, H, DH).transpose(1, 0, 2)
    oh = _attention(qh, kh, vh)
    o = oh.transpose(1, 0, 2).reshape(S, D)
    return _proj_ln(o, res, wo, bo, ln_g, ln_b)


NCHUNK = 24
CH = 256
LG = NCHUNK * CH


def _router_meta_body(x_ref, nw_ref, nb_ref,
                      d1_ref, d2_ref, g1_ref, g2_ref, ce_ref, cv_ref):
    logits = _dotT(x_ref[...], nw_ref[...]) + nb_ref[...]
    iota = jax.lax.broadcasted_iota(jnp.int32, (S, E), 1)
    v1 = jnp.max(logits, axis=1, keepdims=True)
    i1 = jnp.min(jnp.where(logits == v1, iota, E), axis=1, keepdims=True)
    l2 = jnp.where(iota == i1, -jnp.inf, logits)
    v2 = jnp.max(l2, axis=1, keepdims=True)
    i2 = jnp.min(jnp.where(l2 == v2, iota, E), axis=1, keepdims=True)
    e2 = jnp.exp(v2 - v1)
    den = 1.0 + e2
    g1_ref[...] = 1.0 / den
    g2_ref[...] = e2 / den

    oh1 = (iota == i1).astype(jnp.float32)
    oh2 = (iota == i2).astype(jnp.float32)
    r = jax.lax.broadcasted_iota(jnp.int32, (S, S), 0)
    c = jax.lax.broadcasted_iota(jnp.int32, (S, S), 1)
    tril = (c < r).astype(_bf)
    pre1 = jax.lax.dot_general(tril, oh1.astype(_bf), (((1,), (0,)), ((), ())),
                               preferred_element_type=jnp.float32)
    pre2 = jax.lax.dot_general(tril, oh2.astype(_bf), (((1,), (0,)), ((), ())),
                               preferred_element_type=jnp.float32)
    r1 = jnp.sum(pre1 * oh1, axis=1, keepdims=True)
    r2 = jnp.sum(pre2 * oh2, axis=1, keepdims=True)

    cnt1 = jnp.sum(oh1, axis=0, keepdims=True)
    cnt2 = jnp.sum(oh2, axis=0, keepdims=True)
    cnt = cnt1 + cnt2
    nch = jnp.floor((cnt + (CH - 1)) / CH)
    er = jax.lax.broadcasted_iota(jnp.int32, (E, E), 0)
    ec = jax.lax.broadcasted_iota(jnp.int32, (E, E), 1)
    incl = (er <= ec).astype(_bf)
    end = jax.lax.dot_general(nch.astype(_bf), incl, (((1,), (0,)), ((), ())),
                              preferred_element_type=jnp.float32)
    base_rows = (end - nch) * CH

    sel1 = jnp.sum(jnp.where(iota == i1, base_rows, 0.0), axis=1, keepdims=True)
    sel2b = jnp.sum(jnp.where(iota == i2, base_rows, 0.0), axis=1, keepdims=True)
    sel2c = jnp.sum(jnp.where(iota == i2, cnt1, 0.0), axis=1, keepdims=True)
    d1_ref[...] = (sel1 + r1).astype(jnp.int32)
    d2_ref[...] = (sel2b + sel2c + r2).astype(jnp.int32)

    total = jnp.sum(nch)
    cidx = jax.lax.broadcasted_iota(jnp.int32, (1, NCHUNK), 1).astype(jnp.float32)
    cv_ref[...] = (cidx < total).astype(jnp.int32)
    cmin = jnp.minimum(cidx, total - 1.0)
    endb = jnp.broadcast_to(end.reshape(E, 1), (E, NCHUNK))
    ce_ref[...] = jnp.sum((endb <= cmin).astype(jnp.int32), axis=0, keepdims=True)


def _router_meta(x, noise_w, noise_b):
    return pl.pallas_call(
        _router_meta_body,
        grid=(1,),
        in_specs=[pl.BlockSpec((S, D), lambda i: (0, 0)),
                  pl.BlockSpec((E, D), lambda i: (0, 0)),
                  pl.BlockSpec((E,), lambda i: (0,))],
        out_specs=[pl.BlockSpec((S, 1), lambda i: (0, 0)),
                   pl.BlockSpec((S, 1), lambda i: (0, 0)),
                   pl.BlockSpec((S, 1), lambda i: (0, 0)),
                   pl.BlockSpec((S, 1), lambda i: (0, 0)),
                   pl.BlockSpec((1, NCHUNK), lambda i: (0, 0)),
                   pl.BlockSpec((1, NCHUNK), lambda i: (0, 0))],
        out_shape=[jax.ShapeDtypeStruct((S, 1), jnp.int32),
                   jax.ShapeDtypeStruct((S, 1), jnp.int32),
                   jax.ShapeDtypeStruct((S, 1), jnp.float32),
                   jax.ShapeDtypeStruct((S, 1), jnp.float32),
                   jax.ShapeDtypeStruct((1, NCHUNK), jnp.int32),
                   jax.ShapeDtypeStruct((1, NCHUNK), jnp.int32)],
    )(x, noise_w, noise_b)


def _sc_scatter_rows(x, dest):
    import functools as _ft
    from jax.experimental.pallas import tpu_sc as plsc
    info = plsc.get_sparse_core_info()
    NC, NS = info.num_cores, info.num_subcores
    NW = NC * NS
    PPW = (2 * S) // NW
    mesh = plsc.VectorSubcoreMesh(core_axis_name="c", subcore_axis_name="s")

    @_ft.partial(
        pl.kernel, mesh=mesh,
        out_type=jax.ShapeDtypeStruct((LG, D), jnp.float32),
        scratch_types=[pltpu.VMEM((PPW,), jnp.int32),
                       pltpu.VMEM((PPW, D), jnp.float32),
                       pltpu.SemaphoreType.DMA],
    )
    def k(x_hbm, dest_hbm, xg_hbm, idx_v, rows_v, sem):
        wid = jax.lax.axis_index("s") * NC + jax.lax.axis_index("c")
        base = wid * PPW
        tbase = jax.lax.rem(base, S)
        pltpu.sync_copy(dest_hbm.at[pl.ds(base, PPW)], idx_v)
        pltpu.sync_copy(x_hbm.at[pl.ds(tbase, PPW)], rows_v)
        pltpu.async_copy(rows_v, xg_hbm.at[idx_v], sem).wait()

    return k(x, dest)


def _sc_gather_rows(eo, dest):
    import functools as _ft
    from jax.experimental.pallas import tpu_sc as plsc
    info = plsc.get_sparse_core_info()
    NC, NS = info.num_cores, info.num_subcores
    NW = NC * NS
    PPW = (2 * S) // NW
    mesh = plsc.VectorSubcoreMesh(core_axis_name="c", subcore_axis_name="s")

    @_ft.partial(
        pl.kernel, mesh=mesh,
        out_type=jax.ShapeDtypeStruct((2 * S, D), jnp.float32),
        scratch_types=[pltpu.VMEM((PPW,), jnp.int32),
                       pltpu.VMEM((PPW, D), jnp.float32),
                       pltpu.SemaphoreType.DMA],
    )
    def k(eo_hbm, dest_hbm, out_hbm, idx_v, rows_v, sem):
        wid = jax.lax.axis_index("s") * NC + jax.lax.axis_index("c")
        base = wid * PPW
        pltpu.sync_copy(dest_hbm.at[pl.ds(base, PPW)], idx_v)
        pltpu.async_copy(eo_hbm.at[idx_v], rows_v, sem).wait()
        pltpu.sync_copy(rows_v, out_hbm.at[pl.ds(base, PPW)])

    return k(eo, dest)


def _ffn_body(ce_ref, cv_ref, xg_ref, w1_ref, b1_ref, w2_ref, b2_ref, eo_ref):
    c = pl.program_id(0)

    @pl.when(cv_ref[c] == 1)
    def _():
        h = jnp.maximum(_dotT(xg_ref[...], w1_ref[0]) + b1_ref[0], 0.0)
        eo_ref[...] = _dotT(h, w2_ref[0]) + b2_ref[0]


def _ffn_grouped(xg, w1, b1, w2, b2, ce, cv):
    grid_spec = pltpu.PrefetchScalarGridSpec(
        num_scalar_prefetch=2,
        grid=(NCHUNK,),
        in_specs=[
            pl.BlockSpec((CH, D), lambda c, ce, cv: (c, 0)),
            pl.BlockSpec((1, FFN, D), lambda c, ce, cv: (ce[c], 0, 0)),
            pl.BlockSpec((1, 1, FFN), lambda c, ce, cv: (ce[c], 0, 0)),
            pl.BlockSpec((1, D, FFN), lambda c, ce, cv: (ce[c], 0, 0)),
            pl.BlockSpec((1, 1, D), lambda c, ce, cv: (ce[c], 0, 0)),
        ],
        out_specs=pl.BlockSpec((CH, D), lambda c, ce, cv: (c, 0)),
    )
    return pl.pallas_call(
        _ffn_body,
        grid_spec=grid_spec,
        out_shape=jax.ShapeDtypeStruct((LG, D), jnp.float32),
    )(ce, cv, xg, w1, b1.reshape(E, 1, FFN), w2, b2.reshape(E, 1, D))


def _combine_body(x_ref, m1_ref, m2_ref, g1_ref, g2_ref, g_ref, b_ref, y_ref):
    moe = m1_ref[...] * g1_ref[...] + m2_ref[...] * g2_ref[...]
    y_ref[...] = _ln_rows(x_ref[...] + moe, g_ref[...], b_ref[...])


def _combine_ln(x, m1, m2, g1, g2, ln_g, ln_b):
    RB = 256
    row = pl.BlockSpec((RB, D), lambda i: (i, 0))
    col = pl.BlockSpec((RB, 1), lambda i: (i, 0))
    return pl.pallas_call(
        _combine_body,
        grid=(S // RB,),
        in_specs=[row, row, row, col, col,
                  pl.BlockSpec((D,), lambda i: (0,)),
                  pl.BlockSpec((D,), lambda i: (0,))],
        out_specs=row,
        out_shape=jax.ShapeDtypeStruct((S, D), jnp.float32),
    )(x, m1, m2, g1, g2, ln_g, ln_b)


def kernel(tgt, memory, sa_wqkv, sa_bqkv, sa_wo, sa_bo, ma_wqkv, ma_bqkv,
           ma_wo, ma_bo, router_w, router_b, noise_w, noise_b, w1, b1, w2, b2,
           ln1_g, ln1_b, ln2_g, ln2_b, ln3_g, ln3_b):
    x0 = tgt.reshape(S, D)
    mem = memory.reshape(S, D)
    x1 = _mha_block(x0, x0, sa_wqkv, sa_bqkv, sa_wo, sa_bo, x0, ln1_g, ln1_b)
    x2 = _mha_block(x1, mem, ma_wqkv, ma_bqkv, ma_wo, ma_bo, x1, ln2_g, ln2_b)
    d1, d2, g1, g2, ce, cv = _router_meta(x2, noise_w, noise_b)
    dest = jnp.concatenate([d1.reshape(S), d2.reshape(S)])
    xg = _sc_scatter_rows(x2, dest)
    eo = _ffn_grouped(xg, w1, b1, w2, b2, ce.reshape(NCHUNK), cv.reshape(NCHUNK))
    gathered = _sc_gather_rows(eo, dest)
    y = _combine_ln(x2, gathered[:S], gathered[S:], g1, g2, ln3_g, ln3_b)
    return y.reshape(S, 1, D)


# speedup vs baseline: 1.3344x; 1.3344x over previous
import functools

import jax
import jax.numpy as jnp
from jax.experimental import pallas as pl
from jax.experimental.pallas import tpu as pltpu

S, D, H, E, K, FFN = 2048, 768, 12, 8, 2, 2048
DH = D // H

_bf = jnp.bfloat16


def _dotT(a, b):
    return jax.lax.dot_general(
        a.astype(_bf), b.astype(_bf), (((1,), (1,)), ((), ())),
        preferred_element_type=jnp.float32)


def _ln_rows(x, g, b):
    m = jnp.mean(x, axis=-1, keepdims=True)
    v = jnp.mean((x - m) ** 2, axis=-1, keepdims=True)
    return (x - m) / jnp.sqrt(v + 1e-5) * g + b


def _qkv_body(xq_ref, xkv_ref, wq_ref, wk_ref, wv_ref, bq_ref, bk_ref, bv_ref,
              q_ref, k_ref, v_ref):
    q_ref[...] = _dotT(xq_ref[...], wq_ref[...]) + bq_ref[...]
    k_ref[...] = _dotT(xkv_ref[...], wk_ref[...]) + bk_ref[...]
    v_ref[...] = _dotT(xkv_ref[...], wv_ref[...]) + bv_ref[...]


def _qkv_proj(xq, xkv, wqkv, bqkv):
    RB = 256
    grid = (S // RB,)
    wq, wk, wv = wqkv[:D], wqkv[D:2 * D], wqkv[2 * D:]
    bq, bk, bv = bqkv[:D], bqkv[D:2 * D], bqkv[2 * D:]
    full_w = pl.BlockSpec((D, D), lambda i: (0, 0))
    full_b = pl.BlockSpec((D,), lambda i: (0,))
    row = pl.BlockSpec((RB, D), lambda i: (i, 0))
    return pl.pallas_call(
        _qkv_body,
        grid=grid,
        in_specs=[row, row, full_w, full_w, full_w, full_b, full_b, full_b],
        out_specs=[row, row, row],
        out_shape=[jax.ShapeDtypeStruct((S, D), jnp.float32)] * 3,
    )(xq, xkv, wq, wk, wv, bq, bk, bv)


def _attn_one_head(qh, kh, vh):
    s = _dotT(qh, kh)
    m = jnp.max(s, axis=1, keepdims=True)
    p = jnp.exp((s - m) * 0.125)
    l = jnp.sum(p, axis=1, keepdims=True)
    return jax.lax.dot_general(
        (p / l).astype(_bf), vh.astype(_bf), (((1,), (0,)), ((), ())),
        preferred_element_type=jnp.float32)


def _attn_body(q_ref, k_ref, v_ref, o_ref):
    q2, k2, v2 = q_ref[...], k_ref[...], v_ref[...]
    oa = _attn_one_head(q2[:, :DH], k2[:, :DH], v2[:, :DH])
    ob = _attn_one_head(q2[:, DH:], k2[:, DH:], v2[:, DH:])
    o_ref[...] = jnp.concatenate([oa, ob], axis=1)


def _attention(q, k, v):
    QB = 1024
    HP = H // 2
    grid = (HP, S // QB)
    return pl.pallas_call(
        _attn_body,
        grid=grid,
        in_specs=[
            pl.BlockSpec((QB, 2 * DH), lambda hp, qb: (qb, hp)),
            pl.BlockSpec((S, 2 * DH), lambda hp, qb: (0, hp)),
            pl.BlockSpec((S, 2 * DH), lambda hp, qb: (0, hp)),
        ],
        out_specs=pl.BlockSpec((QB, 2 * DH), lambda hp, qb: (qb, hp)),
        out_shape=jax.ShapeDtypeStruct((S, D), jnp.float32),
    )(q, k, v)


def _proj_ln_body(o_ref, res_ref, wo_ref, bo_ref, g_ref, b_ref, y_ref):
    y = _dotT(o_ref[...], wo_ref[...]) + bo_ref[...]
    x = res_ref[...] + y
    y_ref[...] = _ln_rows(x, g_ref[...], b_ref[...])


def _proj_ln(o, res, wo, bo, g, b):
    RB = 256
    row = pl.BlockSpec((RB, D), lambda i: (i, 0))
    return pl.pallas_call(
        _proj_ln_body,
        grid=(S // RB,),
        in_specs=[row, row,
                  pl.BlockSpec((D, D), lambda i: (0, 0)),
                  pl.BlockSpec((D,), lambda i: (0,)),
                  pl.BlockSpec((D,), lambda i: (0,)),
                  pl.BlockSpec((D,), lambda i: (0,))],
        out_specs=row,
        out_shape=jax.ShapeDtypeStruct((S, D), jnp.float32),
    )(o, res, wo, bo, g, b)


def _mha_block(xq, xkv, wqkv, bqkv, wo, bo, res, ln_g, ln_b):
    q, k, v = _qkv_proj(xq, xkv, wqkv, bqkv)
    o = _attention(q, k, v)
    return _proj_ln(o, res, wo, bo, ln_g, ln_b)


NCHUNK = 24
CH = 256
LG = NCHUNK * CH


def _router_meta_body(x_ref, nw_ref, nb_ref,
                      d1_ref, d2_ref, g1_ref, g2_ref, ce_ref, cv_ref):
    logits = _dotT(x_ref[...], nw_ref[...]) + nb_ref[...]
    iota = jax.lax.broadcasted_iota(jnp.int32, (S, E), 1)
    v1 = jnp.max(logits, axis=1, keepdims=True)
    i1 = jnp.min(jnp.where(logits == v1, iota, E), axis=1, keepdims=True)
    l2 = jnp.where(iota == i1, -jnp.inf, logits)
    v2 = jnp.max(l2, axis=1, keepdims=True)
    i2 = jnp.min(jnp.where(l2 == v2, iota, E), axis=1, keepdims=True)
    e2 = jnp.exp(v2 - v1)
    den = 1.0 + e2
    g1_ref[...] = 1.0 / den
    g2_ref[...] = e2 / den

    oh1 = (iota == i1).astype(jnp.float32)
    oh2 = (iota == i2).astype(jnp.float32)
    r = jax.lax.broadcasted_iota(jnp.int32, (S, S), 0)
    c = jax.lax.broadcasted_iota(jnp.int32, (S, S), 1)
    tril = (c < r).astype(_bf)
    pre1 = jax.lax.dot_general(tril, oh1.astype(_bf), (((1,), (0,)), ((), ())),
                               preferred_element_type=jnp.float32)
    pre2 = jax.lax.dot_general(tril, oh2.astype(_bf), (((1,), (0,)), ((), ())),
                               preferred_element_type=jnp.float32)
    r1 = jnp.sum(pre1 * oh1, axis=1, keepdims=True)
    r2 = jnp.sum(pre2 * oh2, axis=1, keepdims=True)

    cnt1 = jnp.sum(oh1, axis=0, keepdims=True)
    cnt2 = jnp.sum(oh2, axis=0, keepdims=True)
    cnt = cnt1 + cnt2
    nch = jnp.floor((cnt + (CH - 1)) / CH)
    er = jax.lax.broadcasted_iota(jnp.int32, (E, E), 0)
    ec = jax.lax.broadcasted_iota(jnp.int32, (E, E), 1)
    incl = (er <= ec).astype(_bf)
    end = jax.lax.dot_general(nch.astype(_bf), incl, (((1,), (0,)), ((), ())),
                              preferred_element_type=jnp.float32)
    base_rows = (end - nch) * CH

    sel1 = jnp.sum(jnp.where(iota == i1, base_rows, 0.0), axis=1, keepdims=True)
    sel2b = jnp.sum(jnp.where(iota == i2, base_rows, 0.0), axis=1, keepdims=True)
    sel2c = jnp.sum(jnp.where(iota == i2, cnt1, 0.0), axis=1, keepdims=True)
    d1_ref[...] = (sel1 + r1).astype(jnp.int32)
    d2_ref[...] = (sel2b + sel2c + r2).astype(jnp.int32)

    total = jnp.sum(nch)
    cidx = jax.lax.broadcasted_iota(jnp.int32, (1, NCHUNK), 1).astype(jnp.float32)
    cv_ref[...] = (cidx < total).astype(jnp.int32)
    cmin = jnp.minimum(cidx, total - 1.0)
    endb = jnp.broadcast_to(end.reshape(E, 1), (E, NCHUNK))
    ce_ref[...] = jnp.sum((endb <= cmin).astype(jnp.int32), axis=0, keepdims=True)


def _router_meta(x, noise_w, noise_b):
    return pl.pallas_call(
        _router_meta_body,
        grid=(1,),
        in_specs=[pl.BlockSpec((S, D), lambda i: (0, 0)),
                  pl.BlockSpec((E, D), lambda i: (0, 0)),
                  pl.BlockSpec((E,), lambda i: (0,))],
        out_specs=[pl.BlockSpec((S, 1), lambda i: (0, 0)),
                   pl.BlockSpec((S, 1), lambda i: (0, 0)),
                   pl.BlockSpec((S, 1), lambda i: (0, 0)),
                   pl.BlockSpec((S, 1), lambda i: (0, 0)),
                   pl.BlockSpec((1, NCHUNK), lambda i: (0, 0)),
                   pl.BlockSpec((1, NCHUNK), lambda i: (0, 0))],
        out_shape=[jax.ShapeDtypeStruct((S, 1), jnp.int32),
                   jax.ShapeDtypeStruct((S, 1), jnp.int32),
                   jax.ShapeDtypeStruct((S, 1), jnp.float32),
                   jax.ShapeDtypeStruct((S, 1), jnp.float32),
                   jax.ShapeDtypeStruct((1, NCHUNK), jnp.int32),
                   jax.ShapeDtypeStruct((1, NCHUNK), jnp.int32)],
    )(x, noise_w, noise_b)


def _sc_scatter_rows(x, dest):
    import functools as _ft
    from jax.experimental.pallas import tpu_sc as plsc
    info = plsc.get_sparse_core_info()
    NC, NS = info.num_cores, info.num_subcores
    NW = NC * NS
    PPW = (2 * S) // NW
    mesh = plsc.VectorSubcoreMesh(core_axis_name="c", subcore_axis_name="s")

    @_ft.partial(
        pl.kernel, mesh=mesh,
        out_type=jax.ShapeDtypeStruct((LG, D), jnp.float32),
        scratch_types=[pltpu.VMEM((PPW,), jnp.int32),
                       pltpu.VMEM((PPW, D), jnp.float32),
                       pltpu.SemaphoreType.DMA],
    )
    def k(x_hbm, dest_hbm, xg_hbm, idx_v, rows_v, sem):
        wid = jax.lax.axis_index("s") * NC + jax.lax.axis_index("c")
        base = wid * PPW
        tbase = jax.lax.rem(base, S)
        pltpu.sync_copy(dest_hbm.at[pl.ds(base, PPW)], idx_v)
        pltpu.sync_copy(x_hbm.at[pl.ds(tbase, PPW)], rows_v)
        pltpu.async_copy(rows_v, xg_hbm.at[idx_v], sem).wait()

    return k(x, dest)


def _sc_gather_rows(eo, dest):
    import functools as _ft
    from jax.experimental.pallas import tpu_sc as plsc
    info = plsc.get_sparse_core_info()
    NC, NS = info.num_cores, info.num_subcores
    NW = NC * NS
    PPW = (2 * S) // NW
    mesh = plsc.VectorSubcoreMesh(core_axis_name="c", subcore_axis_name="s")

    @_ft.partial(
        pl.kernel, mesh=mesh,
        out_type=jax.ShapeDtypeStruct((2 * S, D), jnp.float32),
        scratch_types=[pltpu.VMEM((PPW,), jnp.int32),
                       pltpu.VMEM((PPW, D), jnp.float32),
                       pltpu.SemaphoreType.DMA],
    )
    def k(eo_hbm, dest_hbm, out_hbm, idx_v, rows_v, sem):
        wid = jax.lax.axis_index("s") * NC + jax.lax.axis_index("c")
        base = wid * PPW
        pltpu.sync_copy(dest_hbm.at[pl.ds(base, PPW)], idx_v)
        pltpu.async_copy(eo_hbm.at[idx_v], rows_v, sem).wait()
        pltpu.sync_copy(rows_v, out_hbm.at[pl.ds(base, PPW)])

    return k(eo, dest)


def _ffn_body(ce_ref, cv_ref, xg_ref, w1_ref, b1_ref, w2_ref, b2_ref, eo_ref):
    c = pl.program_id(0)

    @pl.when(cv_ref[c] == 1)
    def _():
        h = jnp.maximum(_dotT(xg_ref[...], w1_ref[0]) + b1_ref[0], 0.0)
        eo_ref[...] = _dotT(h, w2_ref[0]) + b2_ref[0]


def _ffn_grouped(xg, w1, b1, w2, b2, ce, cv):
    grid_spec = pltpu.PrefetchScalarGridSpec(
        num_scalar_prefetch=2,
        grid=(NCHUNK,),
        in_specs=[
            pl.BlockSpec((CH, D), lambda c, ce, cv: (c, 0)),
            pl.BlockSpec((1, FFN, D), lambda c, ce, cv: (ce[c], 0, 0)),
            pl.BlockSpec((1, 1, FFN), lambda c, ce, cv: (ce[c], 0, 0)),
            pl.BlockSpec((1, D, FFN), lambda c, ce, cv: (ce[c], 0, 0)),
            pl.BlockSpec((1, 1, D), lambda c, ce, cv: (ce[c], 0, 0)),
        ],
        out_specs=pl.BlockSpec((CH, D), lambda c, ce, cv: (c, 0)),
    )
    return pl.pallas_call(
        _ffn_body,
        grid_spec=grid_spec,
        out_shape=jax.ShapeDtypeStruct((LG, D), jnp.float32),
    )(ce, cv, xg, w1, b1.reshape(E, 1, FFN), w2, b2.reshape(E, 1, D))


def _combine_body(x_ref, m1_ref, m2_ref, g1_ref, g2_ref, g_ref, b_ref, y_ref):
    moe = m1_ref[...] * g1_ref[...] + m2_ref[...] * g2_ref[...]
    y_ref[...] = _ln_rows(x_ref[...] + moe, g_ref[...], b_ref[...])


def _combine_ln(x, m1, m2, g1, g2, ln_g, ln_b):
    RB = 256
    row = pl.BlockSpec((RB, D), lambda i: (i, 0))
    col = pl.BlockSpec((RB, 1), lambda i: (i, 0))
    return pl.pallas_call(
        _combine_body,
        grid=(S // RB,),
        in_specs=[row, row, row, col, col,
                  pl.BlockSpec((D,), lambda i: (0,)),
                  pl.BlockSpec((D,), lambda i: (0,))],
        out_specs=row,
        out_shape=jax.ShapeDtypeStruct((S, D), jnp.float32),
    )(x, m1, m2, g1, g2, ln_g, ln_b)


def kernel(tgt, memory, sa_wqkv, sa_bqkv, sa_wo, sa_bo, ma_wqkv, ma_bqkv,
           ma_wo, ma_bo, router_w, router_b, noise_w, noise_b, w1, b1, w2, b2,
           ln1_g, ln1_b, ln2_g, ln2_b, ln3_g, ln3_b):
    x0 = tgt.reshape(S, D)
    mem = memory.reshape(S, D)
    x1 = _mha_block(x0, x0, sa_wqkv, sa_bqkv, sa_wo, sa_bo, x0, ln1_g, ln1_b)
    x2 = _mha_block(x1, mem, ma_wqkv, ma_bqkv, ma_wo, ma_bo, x1, ln2_g, ln2_b)
    d1, d2, g1, g2, ce, cv = _router_meta(x2, noise_w, noise_b)
    dest = jnp.concatenate([d1.reshape(S), d2.reshape(S)])
    xg = _sc_scatter_rows(x2, dest)
    eo = _ffn_grouped(xg, w1, b1, w2, b2, ce.reshape(NCHUNK), cv.reshape(NCHUNK))
    gathered = _sc_gather_rows(eo, dest)
    y = _combine_ln(x2, gathered[:S], gathered[S:], g1, g2, ln3_g, ln3_b)
    return y.reshape(S, 1, D)
